# baseline (speedup 1.0000x reference)
import functools

import jax
import jax.numpy as jnp
from jax import lax
from jax.experimental import pallas as pl
from jax.experimental.pallas import tpu as pltpu
from jax.experimental.pallas import tpu_sc as plsc

B = 4096
P = 4096
D = 256
K = 8
DEPTH = 3

NC = 2
NS = 16
NW = NC * NS
GBATCH = 8

NSPLIT = 4
BS = B // NSPLIT

_DN = (((1,), (1,)), ((), ()))


def _pre_body(proto_ref, wroot1_ref, brel1_ref, tab_ref):
    p = proto_ref[...]
    tab_ref[0, :, :D] = p
    hp = brel1_ref[0] + lax.dot_general(p, wroot1_ref[0], _DN,
                                        preferred_element_type=jnp.float32)
    tab_ref[0, :, D:] = jnp.where(hp > 0, hp, 0.2 * hp)


def _precompute(prototypes, W_root1, b_rel1r):
    return pl.pallas_call(
        _pre_body,
        grid=(DEPTH,),
        in_specs=[
            pl.BlockSpec((P, D), lambda d: (0, 0)),
            pl.BlockSpec((1, D, D), lambda d: (d, 0, 0)),
            pl.BlockSpec((1, 1, D), lambda d: (d, 0, 0)),
        ],
        out_specs=pl.BlockSpec((1, P, 2 * D), lambda d: (d, 0, 0)),
        out_shape=jax.ShapeDtypeStruct((DEPTH, P, 2 * D), jnp.float32),
        compiler_params=pltpu.CompilerParams(
            dimension_semantics=("arbitrary",)),
    )(prototypes, W_root1, b_rel1r)


EDGE_R = 256


def _edge_body(fn_ref, x_ref, pnorm_ref, wroot1_ref, m_ref, w_ref,
               froot_ref):
    simt = lax.dot_general(pnorm_ref[...], fn_ref[...], _DN,
                           preferred_element_type=jnp.float32)
    R = EDGE_R
    NCH = P // 128
    cur3 = (-(1.0 - simt)).reshape(NCH, 128, R)
    iota_c3 = lax.broadcasted_iota(jnp.int32, (NCH, 128, R), 0)
    iota_c2 = lax.broadcasted_iota(jnp.int32, (NCH, R), 0)
    iota_l = lax.broadcasted_iota(jnp.int32, (128, R), 0)
    neg_inf = jnp.float32(-jnp.inf)
    cmax = jnp.max(cur3, axis=1)
    idxs, wvals, taken = [], [], []
    for _ in range(K):
        gmax = jnp.max(cmax, axis=0, keepdims=True)
        cstar = jnp.min(jnp.where(cmax == gmax, iota_c2, NCH), axis=0,
                        keepdims=True)
        e = jnp.max(jnp.where(iota_c3 == cstar[:, None, :], cur3, neg_inf),
                    axis=0)
        for ck, pk in taken:
            e = jnp.where((ck == cstar) & (iota_l == pk), neg_inf, e)
        pos = jnp.min(jnp.where(e == gmax, iota_l, 128), axis=0,
                      keepdims=True)
        idxs.append(cstar * 128 + pos)
        wvals.append(jnp.maximum(1.0 - (-gmax), 0.0))
        taken.append((cstar, pos))
        m2 = jnp.max(jnp.where(iota_l == pos, neg_inf, e), axis=0,
                     keepdims=True)
        cmax = jnp.where(iota_c2 == cstar, m2, cmax)
    m_ref[...] = jnp.concatenate(idxs, axis=0)
    w_ref[...] = jnp.concatenate(wvals, axis=0)
    froot_ref[...] = lax.dot_general(x_ref[...], wroot1_ref[...], _DN,
                                     preferred_element_type=jnp.float32)


def _edges(fnorm, features, pnorm, wroot1_d):
    nr = fnorm.shape[0]
    return pl.pallas_call(
        _edge_body,
        grid=(nr // EDGE_R,),
        in_specs=[
            pl.BlockSpec((EDGE_R, D), lambda i: (i, 0)),
            pl.BlockSpec((EDGE_R, D), lambda i: (i, 0)),
            pl.BlockSpec((P, D), lambda i: (0, 0)),
            pl.BlockSpec((D, D), lambda i: (0, 0)),
        ],
        out_specs=[
            pl.BlockSpec((K, EDGE_R), lambda i: (0, i)),
            pl.BlockSpec((K, EDGE_R), lambda i: (0, i)),
            pl.BlockSpec((EDGE_R, D), lambda i: (i, 0)),
        ],
        out_shape=[
            jax.ShapeDtypeStruct((K, nr), jnp.int32),
            jax.ShapeDtypeStruct((K, nr), jnp.float32),
            jax.ShapeDtypeStruct((nr, D), jnp.float32),
        ],
        compiler_params=pltpu.CompilerParams(
            dimension_semantics=("parallel",)),
    )(fnorm, features, pnorm, wroot1_d)


def _sc_gather_body(tab_hbm, idx_hbm, w_hbm, out_hbm, idx_v, w_v, rows0_v,
                    rows1_v, out0_v, out1_v, sem0, sem1, semo0, semo1,
                    rows_per_w):
    nbatch = rows_per_w // GBATCH
    wid = lax.axis_index("s") * NC + lax.axis_index("c")
    base = wid * rows_per_w
    pltpu.sync_copy(idx_hbm.at[pl.ds(base * K, rows_per_w * K)], idx_v)
    pltpu.sync_copy(w_hbm.at[pl.ds(base * K, rows_per_w * K)], w_v)
    rows_bufs = (rows0_v, rows1_v)
    out_bufs = (out0_v, out1_v)
    sems = (sem0, sem1)
    sems_out = (semo0, semo1)

    def start_gather(g, buf, sem):
        pltpu.async_copy(
            tab_hbm.at[idx_v.at[pl.ds(g * GBATCH * K, GBATCH * K)]],
            buf, sem)

    start_gather(0, rows_bufs[0], sems[0])
    start_gather(1, rows_bufs[1], sems[1])

    def compute_batch(g, rows_v, out_v):
        def pair_body(pr, _):
            w16 = w_v[pl.ds(g * (GBATCH * K) + pr * 16, 16)]

            def col_body(j, _):
                acc0 = jnp.zeros((16,), jnp.float32)
                acc1 = jnp.zeros((16,), jnp.float32)
                for k in range(K):
                    acc0 = acc0 + w16[k] * rows_v[pr * 16 + k,
                                                  pl.ds(j * 16, 16)]
                    acc1 = acc1 + w16[K + k] * rows_v[pr * 16 + K + k,
                                                      pl.ds(j * 16, 16)]
                out_v[pr * 2, pl.ds(j * 16, 16)] = acc0
                out_v[pr * 2 + 1, pl.ds(j * 16, 16)] = acc1
                return 0

            return lax.fori_loop(0, (2 * D) // 16, col_body, 0)

        lax.fori_loop(0, GBATCH // 2, pair_body, 0)

    def outer_body(gp, _):
        for b in range(2):
            g = gp * 2 + b
            rows_v = rows_bufs[b]
            out_v = out_bufs[b]
            pltpu.make_async_copy(
                tab_hbm.at[idx_v.at[pl.ds(0, GBATCH * K)]], rows_v,
                sems[b]).wait()
            @pl.when(gp > 0)
            def _():
                pltpu.make_async_copy(
                    out_v, out_hbm.at[pl.ds(base, GBATCH)],
                    sems_out[b]).wait()

            compute_batch(g, rows_v, out_v)
            pltpu.async_copy(
                out_v, out_hbm.at[pl.ds(base + g * GBATCH, GBATCH)],
                sems_out[b])
            @pl.when(g + 2 < nbatch)
            def _():
                start_gather(g + 2, rows_v, sems[b])
        return 0

    lax.fori_loop(0, nbatch // 2, outer_body, 0)
    for b in range(2):
        pltpu.make_async_copy(
            out_bufs[b], out_hbm.at[pl.ds(base, GBATCH)],
            sems_out[b]).wait()


@functools.cache
def _sc_gather_fn(nrows):
    rows_per_w = nrows // NW
    mesh = plsc.VectorSubcoreMesh(
        core_axis_name="c", subcore_axis_name="s", num_cores=NC)
    return pl.kernel(
        functools.partial(_sc_gather_body, rows_per_w=rows_per_w),
        mesh=mesh,
        out_type=jax.ShapeDtypeStruct((nrows, 2 * D), jnp.float32),
        scratch_types=[
            pltpu.VMEM((rows_per_w * K,), jnp.int32),
            pltpu.VMEM((rows_per_w * K,), jnp.float32),
            pltpu.VMEM((GBATCH * K, 2 * D), jnp.float32),
            pltpu.VMEM((GBATCH * K, 2 * D), jnp.float32),
            pltpu.VMEM((GBATCH, 2 * D), jnp.float32),
            pltpu.VMEM((GBATCH, 2 * D), jnp.float32),
            pltpu.SemaphoreType.DMA,
            pltpu.SemaphoreType.DMA,
            pltpu.SemaphoreType.DMA,
            pltpu.SemaphoreType.DMA,
        ],
    )


def _sc_gather(tab_d, idx_flat, w_flat):
    return _sc_gather_fn(idx_flat.shape[0] // K)(tab_d, idx_flat, w_flat)


FIN_R = 1024


def _fin_body(agg_ref, froot_ref, wrel1_ref, brel1_ref, wrel2_ref,
              brel2_ref, wroot2_ref, wlin_ref, blin_ref, out_ref):
    mm = lambda x, w: lax.dot_general(x, w[0], _DN,
                                      preferred_element_type=jnp.float32)
    h = mm(agg_ref[:, :D], wrel1_ref) + brel1_ref[0] + froot_ref[...]
    hf = jnp.where(h > 0, h, 0.2 * h)
    out2 = mm(agg_ref[:, D:], wrel2_ref) + brel2_ref[0] + mm(hf, wroot2_ref)
    out_ref[...] = mm(out2, wlin_ref) + blin_ref[0]


def _finish(agg, froot, wrel1_d, brel1_d, wrel2_d, brel2_d, wroot2_d,
            wlin_d, blin_d):
    nr = agg.shape[0]
    wspec = pl.BlockSpec((1, D, D), lambda i: (0, 0, 0))
    bspec = pl.BlockSpec((1, 1, D), lambda i: (0, 0, 0))
    return pl.pallas_call(
        _fin_body,
        grid=(nr // FIN_R,),
        in_specs=[
            pl.BlockSpec((FIN_R, 2 * D), lambda i: (i, 0)),
            pl.BlockSpec((FIN_R, D), lambda i: (i, 0)),
            wspec, bspec, wspec, bspec, wspec, wspec, bspec,
        ],
        out_specs=pl.BlockSpec((FIN_R, D), lambda i: (i, 0)),
        out_shape=jax.ShapeDtypeStruct((nr, D), jnp.float32),
        compiler_params=pltpu.CompilerParams(
            dimension_semantics=("parallel",)),
    )(agg, froot, wrel1_d, brel1_d, wrel2_d, brel2_d, wroot2_d, wlin_d,
      blin_d)


def kernel(features, prototypes, W_rel1, b_rel1, W_root1, W_rel2, b_rel2,
           W_root2, W_lin, b_lin):
    b_rel1r = b_rel1.reshape(DEPTH, 1, D)
    b_rel2r = b_rel2.reshape(DEPTH, 1, D)
    b_linr = b_lin.reshape(DEPTH, 1, D)
    w3 = lambda w, d: w[d].reshape(1, D, D)
    b3 = lambda b, d: b[d].reshape(1, 1, D)

    tab = _precompute(prototypes, W_root1, b_rel1r)
    pnorm = prototypes / jnp.linalg.norm(prototypes, axis=1, keepdims=True)

    parts = [features[s * BS:(s + 1) * BS] for s in range(NSPLIT)]
    m0 = []
    for d in range(DEPTH):
        m0_parts, new_parts = [], []
        for s in range(NSPLIT):
            x = parts[s]
            fnorm = x / jnp.linalg.norm(x, axis=1, keepdims=True)
            matchest, wvalst, froot = _edges(fnorm, x, pnorm, W_root1[d])
            m0_parts.append(matchest[0])
            agg = _sc_gather(tab[d], matchest.T.reshape(-1),
                             wvalst.T.reshape(-1))
            new_parts.append(
                _finish(agg, froot, w3(W_rel1, d), b3(b_rel1r, d),
                        w3(W_rel2, d), b3(b_rel2r, d), w3(W_root2, d),
                        w3(W_lin, d), b3(b_linr, d)))
        parts = new_parts
        m0.append(jnp.concatenate(m0_parts))
    feats = jnp.concatenate(parts, axis=0)
    return (feats, m0[0], m0[1], m0[2])


# speedup vs baseline: 1.1497x; 1.1497x over previous
import functools

import jax
import jax.numpy as jnp
from jax import lax
from jax.experimental import pallas as pl
from jax.experimental.pallas import tpu as pltpu
from jax.experimental.pallas import tpu_sc as plsc

B = 4096
P = 4096
D = 256
K = 8
DEPTH = 3

NC = 2
NS = 16
NW = NC * NS
GBATCH = 8

NSPLIT = 2
BS = B // NSPLIT

_DN = (((1,), (1,)), ((), ()))


def _pre_body(proto_ref, wroot1_ref, brel1_ref, tab_ref):
    p = proto_ref[...]
    tab_ref[0, :, :D] = p
    hp = brel1_ref[0] + lax.dot_general(p, wroot1_ref[0], _DN,
                                        preferred_element_type=jnp.float32)
    tab_ref[0, :, D:] = jnp.where(hp > 0, hp, 0.2 * hp)


def _precompute(prototypes, W_root1, b_rel1r):
    return pl.pallas_call(
        _pre_body,
        grid=(DEPTH,),
        in_specs=[
            pl.BlockSpec((P, D), lambda d: (0, 0)),
            pl.BlockSpec((1, D, D), lambda d: (d, 0, 0)),
            pl.BlockSpec((1, 1, D), lambda d: (d, 0, 0)),
        ],
        out_specs=pl.BlockSpec((1, P, 2 * D), lambda d: (d, 0, 0)),
        out_shape=jax.ShapeDtypeStruct((DEPTH, P, 2 * D), jnp.float32),
        compiler_params=pltpu.CompilerParams(
            dimension_semantics=("arbitrary",)),
    )(prototypes, W_root1, b_rel1r)


EDGE_R = 256


def _edge_body(fn_ref, x_ref, pnorm_ref, wroot1_ref, m_ref, w_ref,
               froot_ref):
    simt = lax.dot_general(pnorm_ref[...], fn_ref[...], _DN,
                           preferred_element_type=jnp.float32)
    R = EDGE_R
    NCH = P // 128
    cur3 = (-(1.0 - simt)).reshape(NCH, 128, R)
    iota_c3 = lax.broadcasted_iota(jnp.int32, (NCH, 128, R), 0)
    iota_c2 = lax.broadcasted_iota(jnp.int32, (NCH, R), 0)
    iota_l = lax.broadcasted_iota(jnp.int32, (128, R), 0)
    neg_inf = jnp.float32(-jnp.inf)
    cmax = jnp.max(cur3, axis=1)
    idxs, wvals, taken = [], [], []
    for _ in range(K):
        gmax = jnp.max(cmax, axis=0, keepdims=True)
        cstar = jnp.min(jnp.where(cmax == gmax, iota_c2, NCH), axis=0,
                        keepdims=True)
        e = jnp.max(jnp.where(iota_c3 == cstar[:, None, :], cur3, neg_inf),
                    axis=0)
        for ck, pk in taken:
            e = jnp.where((ck == cstar) & (iota_l == pk), neg_inf, e)
        pos = jnp.min(jnp.where(e == gmax, iota_l, 128), axis=0,
                      keepdims=True)
        idxs.append(cstar * 128 + pos)
        wvals.append(jnp.maximum(1.0 - (-gmax), 0.0))
        taken.append((cstar, pos))
        m2 = jnp.max(jnp.where(iota_l == pos, neg_inf, e), axis=0,
                     keepdims=True)
        cmax = jnp.where(iota_c2 == cstar, m2, cmax)
    m_ref[...] = jnp.concatenate(idxs, axis=0)
    w_ref[...] = jnp.concatenate(wvals, axis=0)
    froot_ref[...] = lax.dot_general(x_ref[...], wroot1_ref[...], _DN,
                                     preferred_element_type=jnp.float32)


def _edges(fnorm, features, pnorm, wroot1_d):
    nr = fnorm.shape[0]
    return pl.pallas_call(
        _edge_body,
        grid=(nr // EDGE_R,),
        in_specs=[
            pl.BlockSpec((EDGE_R, D), lambda i: (i, 0)),
            pl.BlockSpec((EDGE_R, D), lambda i: (i, 0)),
            pl.BlockSpec((P, D), lambda i: (0, 0)),
            pl.BlockSpec((D, D), lambda i: (0, 0)),
        ],
        out_specs=[
            pl.BlockSpec((K, EDGE_R), lambda i: (0, i)),
            pl.BlockSpec((K, EDGE_R), lambda i: (0, i)),
            pl.BlockSpec((EDGE_R, D), lambda i: (i, 0)),
        ],
        out_shape=[
            jax.ShapeDtypeStruct((K, nr), jnp.int32),
            jax.ShapeDtypeStruct((K, nr), jnp.float32),
            jax.ShapeDtypeStruct((nr, D), jnp.float32),
        ],
        compiler_params=pltpu.CompilerParams(
            dimension_semantics=("parallel",)),
    )(fnorm, features, pnorm, wroot1_d)


def _sc_gather_body(tab_hbm, idx_hbm, w_hbm, out_hbm, idx_v, w_v, rows0_v,
                    rows1_v, out0_v, out1_v, sem0, sem1, semo0, semo1,
                    rows_per_w):
    nbatch = rows_per_w // GBATCH
    wid = lax.axis_index("s") * NC + lax.axis_index("c")
    base = wid * rows_per_w
    pltpu.sync_copy(idx_hbm.at[pl.ds(base * K, rows_per_w * K)], idx_v)
    pltpu.sync_copy(w_hbm.at[pl.ds(base * K, rows_per_w * K)], w_v)
    rows_bufs = (rows0_v, rows1_v)
    out_bufs = (out0_v, out1_v)
    sems = (sem0, sem1)
    sems_out = (semo0, semo1)

    def start_gather(g, buf, sem):
        pltpu.async_copy(
            tab_hbm.at[idx_v.at[pl.ds(g * GBATCH * K, GBATCH * K)]],
            buf, sem)

    start_gather(0, rows_bufs[0], sems[0])
    start_gather(1, rows_bufs[1], sems[1])

    def compute_batch(g, rows_v, out_v):
        def pair_body(pr, _):
            w16 = w_v[pl.ds(g * (GBATCH * K) + pr * 16, 16)]

            def col_body(j, _):
                acc0 = jnp.zeros((16,), jnp.float32)
                acc1 = jnp.zeros((16,), jnp.float32)
                for k in range(K):
                    acc0 = acc0 + w16[k] * rows_v[pr * 16 + k,
                                                  pl.ds(j * 16, 16)]
                    acc1 = acc1 + w16[K + k] * rows_v[pr * 16 + K + k,
                                                      pl.ds(j * 16, 16)]
                out_v[pr * 2, pl.ds(j * 16, 16)] = acc0
                out_v[pr * 2 + 1, pl.ds(j * 16, 16)] = acc1
                return 0

            return lax.fori_loop(0, (2 * D) // 16, col_body, 0)

        lax.fori_loop(0, GBATCH // 2, pair_body, 0)

    def outer_body(gp, _):
        for b in range(2):
            g = gp * 2 + b
            rows_v = rows_bufs[b]
            out_v = out_bufs[b]
            pltpu.make_async_copy(
                tab_hbm.at[idx_v.at[pl.ds(0, GBATCH * K)]], rows_v,
                sems[b]).wait()
            @pl.when(gp > 0)
            def _():
                pltpu.make_async_copy(
                    out_v, out_hbm.at[pl.ds(base, GBATCH)],
                    sems_out[b]).wait()

            compute_batch(g, rows_v, out_v)
            pltpu.async_copy(
                out_v, out_hbm.at[pl.ds(base + g * GBATCH, GBATCH)],
                sems_out[b])
            @pl.when(g + 2 < nbatch)
            def _():
                start_gather(g + 2, rows_v, sems[b])
        return 0

    lax.fori_loop(0, nbatch // 2, outer_body, 0)
    for b in range(2):
        pltpu.make_async_copy(
            out_bufs[b], out_hbm.at[pl.ds(base, GBATCH)],
            sems_out[b]).wait()


@functools.cache
def _sc_gather_fn(nrows):
    rows_per_w = nrows // NW
    mesh = plsc.VectorSubcoreMesh(
        core_axis_name="c", subcore_axis_name="s", num_cores=NC)
    return pl.kernel(
        functools.partial(_sc_gather_body, rows_per_w=rows_per_w),
        mesh=mesh,
        out_type=jax.ShapeDtypeStruct((nrows, 2 * D), jnp.float32),
        scratch_types=[
            pltpu.VMEM((rows_per_w * K,), jnp.int32),
            pltpu.VMEM((rows_per_w * K,), jnp.float32),
            pltpu.VMEM((GBATCH * K, 2 * D), jnp.float32),
            pltpu.VMEM((GBATCH * K, 2 * D), jnp.float32),
            pltpu.VMEM((GBATCH, 2 * D), jnp.float32),
            pltpu.VMEM((GBATCH, 2 * D), jnp.float32),
            pltpu.SemaphoreType.DMA,
            pltpu.SemaphoreType.DMA,
            pltpu.SemaphoreType.DMA,
            pltpu.SemaphoreType.DMA,
        ],
    )


def _sc_gather(tab_d, idx_flat, w_flat):
    return _sc_gather_fn(idx_flat.shape[0] // K)(tab_d, idx_flat, w_flat)


FIN_R = 1024


def _fin_body(agg_ref, froot_ref, wrel1_ref, brel1_ref, wrel2_ref,
              brel2_ref, wroot2_ref, wlin_ref, blin_ref, out_ref):
    mm = lambda x, w: lax.dot_general(x, w[0], _DN,
                                      preferred_element_type=jnp.float32)
    h = mm(agg_ref[:, :D], wrel1_ref) + brel1_ref[0] + froot_ref[...]
    hf = jnp.where(h > 0, h, 0.2 * h)
    out2 = mm(agg_ref[:, D:], wrel2_ref) + brel2_ref[0] + mm(hf, wroot2_ref)
    out_ref[...] = mm(out2, wlin_ref) + blin_ref[0]


def _finish(agg, froot, wrel1_d, brel1_d, wrel2_d, brel2_d, wroot2_d,
            wlin_d, blin_d):
    nr = agg.shape[0]
    wspec = pl.BlockSpec((1, D, D), lambda i: (0, 0, 0))
    bspec = pl.BlockSpec((1, 1, D), lambda i: (0, 0, 0))
    return pl.pallas_call(
        _fin_body,
        grid=(nr // FIN_R,),
        in_specs=[
            pl.BlockSpec((FIN_R, 2 * D), lambda i: (i, 0)),
            pl.BlockSpec((FIN_R, D), lambda i: (i, 0)),
            wspec, bspec, wspec, bspec, wspec, wspec, bspec,
        ],
        out_specs=pl.BlockSpec((FIN_R, D), lambda i: (i, 0)),
        out_shape=jax.ShapeDtypeStruct((nr, D), jnp.float32),
        compiler_params=pltpu.CompilerParams(
            dimension_semantics=("parallel",)),
    )(agg, froot, wrel1_d, brel1_d, wrel2_d, brel2_d, wroot2_d, wlin_d,
      blin_d)


def kernel(features, prototypes, W_rel1, b_rel1, W_root1, W_rel2, b_rel2,
           W_root2, W_lin, b_lin):
    b_rel1r = b_rel1.reshape(DEPTH, 1, D)
    b_rel2r = b_rel2.reshape(DEPTH, 1, D)
    b_linr = b_lin.reshape(DEPTH, 1, D)
    w3 = lambda w, d: w[d].reshape(1, D, D)
    b3 = lambda b, d: b[d].reshape(1, 1, D)

    tab = _precompute(prototypes, W_root1, b_rel1r)
    pnorm = prototypes / jnp.linalg.norm(prototypes, axis=1, keepdims=True)

    parts = [features[s * BS:(s + 1) * BS] for s in range(NSPLIT)]
    m0 = []
    for d in range(DEPTH):
        m0_parts, new_parts = [], []
        for s in range(NSPLIT):
            x = parts[s]
            fnorm = x / jnp.linalg.norm(x, axis=1, keepdims=True)
            matchest, wvalst, froot = _edges(fnorm, x, pnorm, W_root1[d])
            m0_parts.append(matchest[0])
            agg = _sc_gather(tab[d], matchest.T.reshape(-1),
                             wvalst.T.reshape(-1))
            new_parts.append(
                _finish(agg, froot, w3(W_rel1, d), b3(b_rel1r, d),
                        w3(W_rel2, d), b3(b_rel2r, d), w3(W_root2, d),
                        w3(W_lin, d), b3(b_linr, d)))
        parts = new_parts
        m0.append(jnp.concatenate(m0_parts))
    feats = jnp.concatenate(parts, axis=0)
    return (feats, m0[0], m0[1], m0[2])


# speedup vs baseline: 1.1504x; 1.0006x over previous
import functools

import jax
import jax.numpy as jnp
from jax import lax
from jax.experimental import pallas as pl
from jax.experimental.pallas import tpu as pltpu
from jax.experimental.pallas import tpu_sc as plsc

B = 4096
P = 4096
D = 256
K = 8
DEPTH = 3

NC = 2
NS = 16
NW = NC * NS
GBATCH = 8

NSPLIT = 2
BS = B // NSPLIT

_DN = (((1,), (1,)), ((), ()))


def _pre_body(proto_ref, wroot1_ref, brel1_ref, tab_ref):
    p = proto_ref[...]
    tab_ref[0, :, :D] = p
    hp = brel1_ref[0] + lax.dot_general(p, wroot1_ref[0], _DN,
                                        preferred_element_type=jnp.float32)
    tab_ref[0, :, D:] = jnp.where(hp > 0, hp, 0.2 * hp)


def _precompute(prototypes, W_root1, b_rel1r):
    return pl.pallas_call(
        _pre_body,
        grid=(DEPTH,),
        in_specs=[
            pl.BlockSpec((P, D), lambda d: (0, 0)),
            pl.BlockSpec((1, D, D), lambda d: (d, 0, 0)),
            pl.BlockSpec((1, 1, D), lambda d: (d, 0, 0)),
        ],
        out_specs=pl.BlockSpec((1, P, 2 * D), lambda d: (d, 0, 0)),
        out_shape=jax.ShapeDtypeStruct((DEPTH, P, 2 * D), jnp.float32),
        compiler_params=pltpu.CompilerParams(
            dimension_semantics=("arbitrary",)),
    )(prototypes, W_root1, b_rel1r)


EDGE_R = 256


def _edge_body(fn_ref, x_ref, pnorm_ref, wroot1_ref, m_ref, w_ref,
               froot_ref):
    simt = lax.dot_general(pnorm_ref[...], fn_ref[...], _DN,
                           preferred_element_type=jnp.float32)
    R = EDGE_R
    NCH = P // 128
    cur3 = (-(1.0 - simt)).reshape(NCH, 128, R)
    iota_c3 = lax.broadcasted_iota(jnp.int32, (NCH, 128, R), 0)
    iota_c2 = lax.broadcasted_iota(jnp.int32, (NCH, R), 0)
    iota_l = lax.broadcasted_iota(jnp.int32, (128, R), 0)
    neg_inf = jnp.float32(-jnp.inf)
    cmax = jnp.max(cur3, axis=1)
    idxs, wvals, taken = [], [], []
    for _ in range(K):
        gmax = jnp.max(cmax, axis=0, keepdims=True)
        cstar = jnp.min(jnp.where(cmax == gmax, iota_c2, NCH), axis=0,
                        keepdims=True)
        e = jnp.max(jnp.where(iota_c3 == cstar[:, None, :], cur3, neg_inf),
                    axis=0)
        for ck, pk in taken:
            e = jnp.where((ck == cstar) & (iota_l == pk), neg_inf, e)
        pos = jnp.min(jnp.where(e == gmax, iota_l, 128), axis=0,
                      keepdims=True)
        idxs.append(cstar * 128 + pos)
        wvals.append(jnp.maximum(1.0 - (-gmax), 0.0))
        taken.append((cstar, pos))
        m2 = jnp.max(jnp.where(iota_l == pos, neg_inf, e), axis=0,
                     keepdims=True)
        cmax = jnp.where(iota_c2 == cstar, m2, cmax)
    m_ref[...] = jnp.concatenate(idxs, axis=0)
    w_ref[...] = jnp.concatenate(wvals, axis=0)
    froot_ref[...] = lax.dot_general(x_ref[...], wroot1_ref[...], _DN,
                                     preferred_element_type=jnp.float32)


def _edges(fnorm, features, pnorm, wroot1_d):
    nr = fnorm.shape[0]
    return pl.pallas_call(
        _edge_body,
        grid=(nr // EDGE_R,),
        in_specs=[
            pl.BlockSpec((EDGE_R, D), lambda i: (i, 0)),
            pl.BlockSpec((EDGE_R, D), lambda i: (i, 0)),
            pl.BlockSpec((P, D), lambda i: (0, 0)),
            pl.BlockSpec((D, D), lambda i: (0, 0)),
        ],
        out_specs=[
            pl.BlockSpec((K, EDGE_R), lambda i: (0, i)),
            pl.BlockSpec((K, EDGE_R), lambda i: (0, i)),
            pl.BlockSpec((EDGE_R, D), lambda i: (i, 0)),
        ],
        out_shape=[
            jax.ShapeDtypeStruct((K, nr), jnp.int32),
            jax.ShapeDtypeStruct((K, nr), jnp.float32),
            jax.ShapeDtypeStruct((nr, D), jnp.float32),
        ],
        compiler_params=pltpu.CompilerParams(
            dimension_semantics=("parallel",)),
    )(fnorm, features, pnorm, wroot1_d)


def _sc_gather_body(tab_hbm, idx_hbm, w_hbm, out_hbm, idx_v, w_v, rows0_v,
                    rows1_v, out0_v, out1_v, sem0, sem1, semo0, semo1,
                    rows_per_w):
    nbatch = rows_per_w // GBATCH
    wid = lax.axis_index("s") * NC + lax.axis_index("c")
    base = wid * rows_per_w
    pltpu.sync_copy(idx_hbm.at[pl.ds(base * K, rows_per_w * K)], idx_v)
    pltpu.sync_copy(w_hbm.at[pl.ds(base * K, rows_per_w * K)], w_v)
    rows_bufs = (rows0_v, rows1_v)
    out_bufs = (out0_v, out1_v)
    sems = (sem0, sem1)
    sems_out = (semo0, semo1)

    def start_gather(g, buf, sem):
        pltpu.async_copy(
            tab_hbm.at[idx_v.at[pl.ds(g * GBATCH * K, GBATCH * K)]],
            buf, sem)

    start_gather(0, rows_bufs[0], sems[0])
    start_gather(1, rows_bufs[1], sems[1])

    def compute_batch(g, rows_v, out_v):
        def pair_body(pr, _):
            w16 = w_v[pl.ds(g * (GBATCH * K) + pr * 16, 16)]
            ws = [w16[k] for k in range(16)]

            def col_body(jb, _):
                for jj in range(4):
                    j = jb * 4 + jj
                    acc0 = jnp.zeros((16,), jnp.float32)
                    acc1 = jnp.zeros((16,), jnp.float32)
                    for k in range(K):
                        acc0 = acc0 + ws[k] * rows_v[pr * 16 + k,
                                                     pl.ds(j * 16, 16)]
                        acc1 = acc1 + ws[K + k] * rows_v[pr * 16 + K + k,
                                                         pl.ds(j * 16, 16)]
                    out_v[pr * 2, pl.ds(j * 16, 16)] = acc0
                    out_v[pr * 2 + 1, pl.ds(j * 16, 16)] = acc1
                return 0

            return lax.fori_loop(0, (2 * D) // 64, col_body, 0)

        lax.fori_loop(0, GBATCH // 2, pair_body, 0)

    def outer_body(gp, _):
        for b in range(2):
            g = gp * 2 + b
            rows_v = rows_bufs[b]
            out_v = out_bufs[b]
            pltpu.make_async_copy(
                tab_hbm.at[idx_v.at[pl.ds(0, GBATCH * K)]], rows_v,
                sems[b]).wait()
            @pl.when(gp > 0)
            def _():
                pltpu.make_async_copy(
                    out_v, out_hbm.at[pl.ds(base, GBATCH)],
                    sems_out[b]).wait()

            compute_batch(g, rows_v, out_v)
            pltpu.async_copy(
                out_v, out_hbm.at[pl.ds(base + g * GBATCH, GBATCH)],
                sems_out[b])
            @pl.when(g + 2 < nbatch)
            def _():
                start_gather(g + 2, rows_v, sems[b])
        return 0

    lax.fori_loop(0, nbatch // 2, outer_body, 0)
    for b in range(2):
        pltpu.make_async_copy(
            out_bufs[b], out_hbm.at[pl.ds(base, GBATCH)],
            sems_out[b]).wait()


@functools.cache
def _sc_gather_fn(nrows):
    rows_per_w = nrows // NW
    mesh = plsc.VectorSubcoreMesh(
        core_axis_name="c", subcore_axis_name="s", num_cores=NC)
    return pl.kernel(
        functools.partial(_sc_gather_body, rows_per_w=rows_per_w),
        mesh=mesh,
        out_type=jax.ShapeDtypeStruct((nrows, 2 * D), jnp.float32),
        scratch_types=[
            pltpu.VMEM((rows_per_w * K,), jnp.int32),
            pltpu.VMEM((rows_per_w * K,), jnp.float32),
            pltpu.VMEM((GBATCH * K, 2 * D), jnp.float32),
            pltpu.VMEM((GBATCH * K, 2 * D), jnp.float32),
            pltpu.VMEM((GBATCH, 2 * D), jnp.float32),
            pltpu.VMEM((GBATCH, 2 * D), jnp.float32),
            pltpu.SemaphoreType.DMA,
            pltpu.SemaphoreType.DMA,
            pltpu.SemaphoreType.DMA,
            pltpu.SemaphoreType.DMA,
        ],
    )


def _sc_gather(tab_d, idx_flat, w_flat):
    return _sc_gather_fn(idx_flat.shape[0] // K)(tab_d, idx_flat, w_flat)


FIN_R = 1024


def _fin_body(agg_ref, froot_ref, wrel1_ref, brel1_ref, wrel2_ref,
              brel2_ref, wroot2_ref, wlin_ref, blin_ref, out_ref):
    mm = lambda x, w: lax.dot_general(x, w[0], _DN,
                                      preferred_element_type=jnp.float32)
    h = mm(agg_ref[:, :D], wrel1_ref) + brel1_ref[0] + froot_ref[...]
    hf = jnp.where(h > 0, h, 0.2 * h)
    out2 = mm(agg_ref[:, D:], wrel2_ref) + brel2_ref[0] + mm(hf, wroot2_ref)
    out_ref[...] = mm(out2, wlin_ref) + blin_ref[0]


def _finish(agg, froot, wrel1_d, brel1_d, wrel2_d, brel2_d, wroot2_d,
            wlin_d, blin_d):
    nr = agg.shape[0]
    wspec = pl.BlockSpec((1, D, D), lambda i: (0, 0, 0))
    bspec = pl.BlockSpec((1, 1, D), lambda i: (0, 0, 0))
    return pl.pallas_call(
        _fin_body,
        grid=(nr // FIN_R,),
        in_specs=[
            pl.BlockSpec((FIN_R, 2 * D), lambda i: (i, 0)),
            pl.BlockSpec((FIN_R, D), lambda i: (i, 0)),
            wspec, bspec, wspec, bspec, wspec, wspec, bspec,
        ],
        out_specs=pl.BlockSpec((FIN_R, D), lambda i: (i, 0)),
        out_shape=jax.ShapeDtypeStruct((nr, D), jnp.float32),
        compiler_params=pltpu.CompilerParams(
            dimension_semantics=("parallel",)),
    )(agg, froot, wrel1_d, brel1_d, wrel2_d, brel2_d, wroot2_d, wlin_d,
      blin_d)


def kernel(features, prototypes, W_rel1, b_rel1, W_root1, W_rel2, b_rel2,
           W_root2, W_lin, b_lin):
    b_rel1r = b_rel1.reshape(DEPTH, 1, D)
    b_rel2r = b_rel2.reshape(DEPTH, 1, D)
    b_linr = b_lin.reshape(DEPTH, 1, D)
    w3 = lambda w, d: w[d].reshape(1, D, D)
    b3 = lambda b, d: b[d].reshape(1, 1, D)

    tab = _precompute(prototypes, W_root1, b_rel1r)
    pnorm = prototypes / jnp.linalg.norm(prototypes, axis=1, keepdims=True)

    parts = [features[s * BS:(s + 1) * BS] for s in range(NSPLIT)]
    m0 = []
    for d in range(DEPTH):
        m0_parts, new_parts = [], []
        for s in range(NSPLIT):
            x = parts[s]
            fnorm = x / jnp.linalg.norm(x, axis=1, keepdims=True)
            matchest, wvalst, froot = _edges(fnorm, x, pnorm, W_root1[d])
            m0_parts.append(matchest[0])
            agg = _sc_gather(tab[d], matchest.T.reshape(-1),
                             wvalst.T.reshape(-1))
            new_parts.append(
                _finish(agg, froot, w3(W_rel1, d), b3(b_rel1r, d),
                        w3(W_rel2, d), b3(b_rel2r, d), w3(W_root2, d),
                        w3(W_lin, d), b3(b_linr, d)))
        parts = new_parts
        m0.append(jnp.concatenate(m0_parts))
    feats = jnp.concatenate(parts, axis=0)
    return (feats, m0[0], m0[1], m0[2])
